# Initial kernel scaffold; baseline (speedup 1.0000x reference)
import jax
import jax.numpy as jnp
from jax.experimental import pallas as pl


def kernel(features, edge_index, neg_edge_index, W1, b1, W2, b2):
    raise NotImplementedError("write your pallas kernel here")


# speedup vs baseline: 10.1321x; 10.1321x over previous
import functools

import jax
import jax.numpy as jnp
from jax import lax
from jax.experimental import pallas as pl
from jax.experimental.pallas import tpu as pltpu
from jax.experimental.pallas import tpu_sc as plsc

N = 10000
NPAD = 10240
E = 320000
NNEG = 50000
NNEG_PAD = 51200
F_IN = 128
H = 64
NC = 2
NS = 16
NW = NC * NS
RPT = NPAD // NS

CHUNK_D = 2000
CHUNK_E = 400
CHUNK_S = 400

_mesh = plsc.VectorSubcoreMesh(core_axis_name="c", subcore_axis_name="s")
_sc_params = pltpu.CompilerParams(use_tc_tiling_on_sc=False)
_sc_params_nl = pltpu.CompilerParams(use_tc_tiling_on_sc=False,
                                    needs_layout_passes=False)


def _sc_degree_body(col_hbm, out_hbm, deg_sp, cidx_v, ones_v):
    cid = lax.axis_index("c")
    sid = lax.axis_index("s")

    def zfill(i, _):
        ones_v[pl.ds(i * 16, 16)] = jnp.zeros((16,), jnp.float32)
        return 0

    lax.fori_loop(0, RPT // 16, zfill, 0)
    pltpu.sync_copy(ones_v.at[pl.ds(0, RPT)], deg_sp.at[pl.ds(sid * RPT, RPT)])

    def fill(i, _):
        ones_v[pl.ds(i * 16, 16)] = jnp.ones((16,), jnp.float32)
        return 0

    lax.fori_loop(0, CHUNK_D // 16, fill, 0)
    plsc.subcore_barrier()

    epw = E // NW
    base = (cid * NS + sid) * epw

    def body(k, _):
        pltpu.sync_copy(col_hbm.at[pl.ds(base + k * CHUNK_D, CHUNK_D)], cidx_v)
        pltpu.sync_copy(ones_v, deg_sp.at[cidx_v], add=True)
        return 0

    lax.fori_loop(0, epw // CHUNK_D, body, 0)
    plsc.subcore_barrier()
    pltpu.sync_copy(deg_sp.at[pl.ds(sid * RPT, RPT)],
                    out_hbm.at[cid, pl.ds(sid * RPT, RPT)])


def _sc_degree(col):
    k = functools.partial(
        pl.kernel,
        out_type=jax.ShapeDtypeStruct((NC, NPAD), jnp.float32),
        mesh=_mesh,
        compiler_params=_sc_params,
        scratch_types=[
            pltpu.VMEM_SHARED((NPAD,), jnp.float32),
            pltpu.VMEM((CHUNK_D,), jnp.int32),
            pltpu.VMEM((CHUNK_D,), jnp.float32),
        ],
    )(_sc_degree_body)
    return k(col)


def _sc_edge_body(y_hbm, row_hbm, col_hbm, out_hbm,
                  y_sp, acc_sp, ridx_v, cidx_v, rows_v):
    cid = lax.axis_index("c")
    sid = lax.axis_index("s")

    def zfill(i, _):
        r = i // 4
        j = i % 4
        rows_v[r, pl.ds(j * 16, 16)] = jnp.zeros((16,), jnp.float32)
        return 0

    lax.fori_loop(0, CHUNK_E * 4, zfill, 0)
    pltpu.sync_copy(rows_v, acc_sp.at[pl.ds(sid * RPT, CHUNK_E)])
    pltpu.sync_copy(rows_v.at[pl.ds(0, RPT - CHUNK_E)],
                    acc_sp.at[pl.ds(sid * RPT + CHUNK_E, RPT - CHUNK_E)])
    pltpu.sync_copy(y_hbm.at[pl.ds(sid * RPT, RPT)],
                    y_sp.at[pl.ds(sid * RPT, RPT)])
    plsc.subcore_barrier()

    epw = E // NW
    base = (cid * NS + sid) * epw

    def body(k, _):
        s = base + k * CHUNK_E
        pltpu.sync_copy(row_hbm.at[pl.ds(s, CHUNK_E)], ridx_v)
        pltpu.sync_copy(col_hbm.at[pl.ds(s, CHUNK_E)], cidx_v)
        pltpu.sync_copy(y_sp.at[ridx_v], rows_v)
        pltpu.sync_copy(rows_v, acc_sp.at[cidx_v], add=True)
        return 0

    lax.fori_loop(0, epw // CHUNK_E, body, 0)
    plsc.subcore_barrier()
    pltpu.sync_copy(acc_sp.at[pl.ds(sid * RPT, RPT)],
                    out_hbm.at[cid, pl.ds(sid * RPT, RPT)])


def _sc_edge_pass(y_pad, row, col):
    k = functools.partial(
        pl.kernel,
        out_type=jax.ShapeDtypeStruct((NC, NPAD, H), jnp.float32),
        mesh=_mesh,
        compiler_params=_sc_params,
        scratch_types=[
            pltpu.VMEM_SHARED((NPAD, H), jnp.float32),
            pltpu.VMEM_SHARED((NPAD, H), jnp.float32),
            pltpu.VMEM((CHUNK_E,), jnp.int32),
            pltpu.VMEM((CHUNK_E,), jnp.int32),
            pltpu.VMEM((CHUNK_E, H), jnp.float32),
        ],
    )(_sc_edge_body)
    return k(y_pad, row, col)


def _sc_score_body(rep_hbm, row_hbm, col_hbm, nr_hbm, nc_hbm, out_hbm,
                   rep_sp, aidx_v, bidx_v, arows_v, brows_v, res_v):
    cid = lax.axis_index("c")
    sid = lax.axis_index("s")
    w = cid * NS + sid

    pltpu.sync_copy(rep_hbm.at[pl.ds(sid * RPT, RPT)],
                    rep_sp.at[pl.ds(sid * RPT, RPT)])
    plsc.subcore_barrier()

    def make_loop(nchunks, base0, r_hbm, c_hbm, sub_one):
        def chunk_body(k, carry):
            ssum, scnt = carry
            s = base0 + k * CHUNK_S
            pltpu.sync_copy(r_hbm.at[pl.ds(s, CHUNK_S)], aidx_v)
            pltpu.sync_copy(c_hbm.at[pl.ds(s, CHUNK_S)], bidx_v)
            pltpu.sync_copy(rep_sp.at[aidx_v], arows_v)
            pltpu.sync_copy(rep_sp.at[bidx_v], brows_v)

            def grp(g, c2):
                s2, c2c = c2
                rows0 = g * 16 + lax.iota(jnp.int32, 16)
                acc = jnp.zeros((16,), jnp.float32)
                for f in range(H):
                    colsf = jnp.full((16,), f, jnp.int32)
                    a = plsc.load_gather(arows_v, [rows0, colsf])
                    b = plsc.load_gather(brows_v, [rows0, colsf])
                    acc = acc + a * b
                ra = plsc.load_gather(aidx_v, [rows0])
                rb = plsc.load_gather(bidx_v, [rows0])
                m = ra < rb
                t = acc - 1.0 if sub_one else acc
                s2 = s2 + jnp.where(m, t * t, jnp.zeros((16,), jnp.float32))
                c2c = c2c + jnp.where(m, jnp.ones((16,), jnp.float32),
                                      jnp.zeros((16,), jnp.float32))
                return (s2, c2c)

            return lax.fori_loop(0, CHUNK_S // 16, grp, (ssum, scnt))

        init = (jnp.zeros((16,), jnp.float32), jnp.zeros((16,), jnp.float32))
        return lax.fori_loop(0, nchunks, chunk_body, init)

    ppw = E // NW
    npw = NNEG_PAD // NW
    ps, pc = make_loop(ppw // CHUNK_S, w * ppw, row_hbm, col_hbm, True)
    qs, qc = make_loop(npw // CHUNK_S, w * npw, nr_hbm, nc_hbm, False)
    res_v[pl.ds(0, 16)] = ps
    res_v[pl.ds(16, 16)] = pc
    res_v[pl.ds(32, 16)] = qs
    res_v[pl.ds(48, 16)] = qc
    pltpu.sync_copy(res_v, out_hbm.at[w])


def _sc_score(rep_pad, row, col, nr_pad, nc_pad):
    k = functools.partial(
        pl.kernel,
        out_type=jax.ShapeDtypeStruct((NW, 4 * 16), jnp.float32),
        mesh=_mesh,
        compiler_params=_sc_params_nl,
        scratch_types=[
            pltpu.VMEM_SHARED((NPAD, H), jnp.float32),
            pltpu.VMEM((CHUNK_S,), jnp.int32),
            pltpu.VMEM((CHUNK_S,), jnp.int32),
            pltpu.VMEM((CHUNK_S, H), jnp.float32),
            pltpu.VMEM((CHUNK_S, H), jnp.float32),
            pltpu.VMEM((4 * 16,), jnp.float32),
        ],
    )(_sc_score_body)
    return k(rep_pad, row, col, nr_pad, nc_pad)


def _mm1_body(x_ref, w_ref, o_ref):
    o_ref[...] = jnp.dot(x_ref[...], w_ref[...],
                         preferred_element_type=jnp.float32)


def _tc_mm1(features, W1):
    return pl.pallas_call(
        _mm1_body,
        out_shape=jax.ShapeDtypeStruct((N, H), jnp.float32),
    )(features, W1)


def _prep_body(degt_ref, xw_ref, dinv_ref, y_ref):
    d = degt_ref[...]
    deg = d[:, 0:1] + d[:, 1:2] + 1.0
    dinv = lax.rsqrt(deg)
    dinv_ref[...] = dinv
    y_ref[0:N, :] = dinv[0:N] * xw_ref[...]
    y_ref[N:NPAD, :] = jnp.zeros((NPAD - N, H), jnp.float32)


def _tc_prep(degt, xw1):
    return pl.pallas_call(
        _prep_body,
        out_shape=[jax.ShapeDtypeStruct((NPAD, 1), jnp.float32),
                   jax.ShapeDtypeStruct((NPAD, H), jnp.float32)],
    )(degt, xw1)


def _mid_body(acc_ref, y_ref, dinv_ref, b1_ref, w2_ref, o_ref):
    dinv = dinv_ref[...]
    s = acc_ref[0] + acc_ref[1] + y_ref[...]
    h = jnp.maximum(dinv * s + b1_ref[...], 0.0)
    xw2 = jnp.dot(h, w2_ref[...], preferred_element_type=jnp.float32)
    y2 = dinv * xw2
    o_ref[0:N, :] = y2[0:N]
    o_ref[N:NPAD, :] = jnp.zeros((NPAD - N, H), jnp.float32)


def _tc_mid(accp1, y1p, dinvp, b1, W2):
    return pl.pallas_call(
        _mid_body,
        out_shape=jax.ShapeDtypeStruct((NPAD, H), jnp.float32),
    )(accp1, y1p, dinvp, b1, W2)


def _final_body(acc_ref, y_ref, dinv_ref, b2_ref, o_ref):
    s = acc_ref[0, 0:N] + acc_ref[1, 0:N] + y_ref[0:N]
    o_ref[...] = dinv_ref[0:N] * s + b2_ref[...]


def _tc_final(accp2, y2p, dinvp, b2):
    return pl.pallas_call(
        _final_body,
        out_shape=jax.ShapeDtypeStruct((N, H), jnp.float32),
    )(accp2, y2p, dinvp, b2)


def _loss_body(parts_ref, o_ref):
    p = parts_ref[...]
    pos_sq = jnp.sum(p[:, 0:16])
    pos_c = jnp.sum(p[:, 16:32])
    neg_sq = jnp.sum(p[:, 32:48])
    neg_c = jnp.sum(p[:, 48:64])
    rec = (neg_sq + pos_sq) * jnp.float32(N) / (neg_c + pos_c)
    o_ref[...] = jnp.broadcast_to(rec, (1, 1))


def _tc_loss(parts):
    return pl.pallas_call(
        _loss_body,
        out_shape=jax.ShapeDtypeStruct((1, 1), jnp.float32),
    )(parts)


def kernel(features, edge_index, neg_edge_index, W1, b1, W2, b2):
    assert features.shape == (N, F_IN)
    assert edge_index.shape == (2, E)
    assert neg_edge_index.shape == (2, NNEG)

    row = edge_index[0]
    col = edge_index[1]
    nr = neg_edge_index[0]
    nc = neg_edge_index[1]
    zpad = jnp.zeros((NNEG_PAD - NNEG,), jnp.int32)
    nr_p = jnp.concatenate([nr, zpad])
    nc_p = jnp.concatenate([nc, zpad])

    xw1 = _tc_mm1(features, W1)
    degp = _sc_degree(col)
    degt = jnp.transpose(degp)
    dinvp, y1p = _tc_prep(degt, xw1)
    accp1 = _sc_edge_pass(y1p, row, col)
    y2p = _tc_mid(accp1, y1p, dinvp, b1, W2)
    accp2 = _sc_edge_pass(y2p, row, col)
    rep = _tc_final(accp2, y2p, dinvp, b2)
    rep_pad = jnp.concatenate(
        [rep, jnp.zeros((NPAD - N, H), jnp.float32)], axis=0)
    parts = _sc_score(rep_pad, row, col, nr_p, nc_p)
    rec_loss = _tc_loss(parts)[0, 0]
    return (rep, rec_loss)


# speedup vs baseline: 19.7053x; 1.9448x over previous
import functools

import jax
import jax.numpy as jnp
from jax import lax
from jax.experimental import pallas as pl
from jax.experimental.pallas import tpu as pltpu
from jax.experimental.pallas import tpu_sc as plsc

N = 10000
NPAD = 10240
E = 320000
NNEG = 50000
NNEG_PAD = 51200
F_IN = 128
H = 64
NC = 2
NS = 16
NW = NC * NS
RPT = NPAD // NS

CHUNK_D = 2000
CHUNK_E = 400
CHUNK_S = 400

_mesh = plsc.VectorSubcoreMesh(core_axis_name="c", subcore_axis_name="s")
_sc_params = pltpu.CompilerParams(use_tc_tiling_on_sc=False)
_sc_params_nl = pltpu.CompilerParams(use_tc_tiling_on_sc=False,
                                    needs_layout_passes=False)


def _sc_degree_body(col_hbm, out_hbm, deg_sp, cidx_v, ones_v):
    cid = lax.axis_index("c")
    sid = lax.axis_index("s")

    def zfill(i, _):
        ones_v[pl.ds(i * 16, 16)] = jnp.zeros((16,), jnp.float32)
        return 0

    lax.fori_loop(0, RPT // 16, zfill, 0)
    pltpu.sync_copy(ones_v.at[pl.ds(0, RPT)], deg_sp.at[pl.ds(sid * RPT, RPT)])

    def fill(i, _):
        ones_v[pl.ds(i * 16, 16)] = jnp.ones((16,), jnp.float32)
        return 0

    lax.fori_loop(0, CHUNK_D // 16, fill, 0)
    plsc.subcore_barrier()

    epw = E // NW
    base = (cid * NS + sid) * epw

    def body(k, _):
        pltpu.sync_copy(col_hbm.at[pl.ds(base + k * CHUNK_D, CHUNK_D)], cidx_v)
        pltpu.sync_copy(ones_v, deg_sp.at[cidx_v], add=True)
        return 0

    lax.fori_loop(0, epw // CHUNK_D, body, 0)
    plsc.subcore_barrier()
    pltpu.sync_copy(deg_sp.at[pl.ds(sid * RPT, RPT)],
                    out_hbm.at[cid, pl.ds(sid * RPT, RPT)])


def _sc_degree(col):
    k = functools.partial(
        pl.kernel,
        out_type=jax.ShapeDtypeStruct((NC, NPAD), jnp.float32),
        mesh=_mesh,
        compiler_params=_sc_params,
        scratch_types=[
            pltpu.VMEM_SHARED((NPAD,), jnp.float32),
            pltpu.VMEM((CHUNK_D,), jnp.int32),
            pltpu.VMEM((CHUNK_D,), jnp.float32),
        ],
    )(_sc_degree_body)
    return k(col)


def _sc_edge_body(y_hbm, row_hbm, col_hbm, out_hbm,
                  y_sp, acc_sp, ridx_v, cidx_v, rows_v):
    cid = lax.axis_index("c")
    sid = lax.axis_index("s")

    def zfill(i, _):
        r = i // 4
        j = i % 4
        rows_v[r, pl.ds(j * 16, 16)] = jnp.zeros((16,), jnp.float32)
        return 0

    lax.fori_loop(0, CHUNK_E * 4, zfill, 0)
    pltpu.sync_copy(rows_v, acc_sp.at[pl.ds(sid * RPT, CHUNK_E)])
    pltpu.sync_copy(rows_v.at[pl.ds(0, RPT - CHUNK_E)],
                    acc_sp.at[pl.ds(sid * RPT + CHUNK_E, RPT - CHUNK_E)])
    pltpu.sync_copy(y_hbm.at[pl.ds(sid * RPT, RPT)],
                    y_sp.at[pl.ds(sid * RPT, RPT)])
    plsc.subcore_barrier()

    epw = E // NW
    base = (cid * NS + sid) * epw

    def body(k, _):
        s = base + k * CHUNK_E
        pltpu.sync_copy(row_hbm.at[pl.ds(s, CHUNK_E)], ridx_v)
        pltpu.sync_copy(col_hbm.at[pl.ds(s, CHUNK_E)], cidx_v)
        pltpu.sync_copy(y_sp.at[ridx_v], rows_v)
        pltpu.sync_copy(rows_v, acc_sp.at[cidx_v], add=True)
        return 0

    lax.fori_loop(0, epw // CHUNK_E, body, 0)
    plsc.subcore_barrier()
    pltpu.sync_copy(acc_sp.at[pl.ds(sid * RPT, RPT)],
                    out_hbm.at[cid, pl.ds(sid * RPT, RPT)])


def _sc_edge_pass(y_pad, row, col):
    k = functools.partial(
        pl.kernel,
        out_type=jax.ShapeDtypeStruct((NC, NPAD, H), jnp.float32),
        mesh=_mesh,
        compiler_params=_sc_params,
        scratch_types=[
            pltpu.VMEM_SHARED((NPAD, H), jnp.float32),
            pltpu.VMEM_SHARED((NPAD, H), jnp.float32),
            pltpu.VMEM((CHUNK_E,), jnp.int32),
            pltpu.VMEM((CHUNK_E,), jnp.int32),
            pltpu.VMEM((CHUNK_E, H), jnp.float32),
        ],
    )(_sc_edge_body)
    return k(y_pad, row, col)


def _sc_score_body(rep_hbm, row_hbm, col_hbm, nr_hbm, nc_hbm, out_hbm,
                   rep_sp, aidx_v, bidx_v, arows_v, brows_v, res_v):
    cid = lax.axis_index("c")
    sid = lax.axis_index("s")
    w = cid * NS + sid

    pltpu.sync_copy(rep_hbm.at[pl.ds(sid * RPT, RPT)],
                    rep_sp.at[pl.ds(sid * RPT, RPT)])
    plsc.subcore_barrier()

    def make_loop(nchunks, base0, r_hbm, c_hbm, sub_one):
        def chunk_body(k, carry):
            ssum, scnt = carry
            s = base0 + k * CHUNK_S
            pltpu.sync_copy(r_hbm.at[pl.ds(s, CHUNK_S)], aidx_v)
            pltpu.sync_copy(c_hbm.at[pl.ds(s, CHUNK_S)], bidx_v)
            pltpu.sync_copy(rep_sp.at[aidx_v], arows_v)
            pltpu.sync_copy(rep_sp.at[bidx_v], brows_v)

            def grp(g, c2):
                s2, c2c = c2
                lane = lax.iota(jnp.int32, 16)
                rows0 = g * 16 + lane
                accs = [jnp.zeros((16,), jnp.float32) for _ in range(4)]
                for f in range(H):
                    colsf = jnp.bitwise_and(f + lane, H - 1)
                    a = plsc.load_gather(arows_v, [rows0, colsf])
                    b = plsc.load_gather(brows_v, [rows0, colsf])
                    accs[f % 4] = accs[f % 4] + a * b
                acc = (accs[0] + accs[1]) + (accs[2] + accs[3])
                ra = plsc.load_gather(aidx_v, [rows0])
                rb = plsc.load_gather(bidx_v, [rows0])
                m = ra < rb
                t = acc - 1.0 if sub_one else acc
                s2 = s2 + jnp.where(m, t * t, jnp.zeros((16,), jnp.float32))
                c2c = c2c + jnp.where(m, jnp.ones((16,), jnp.float32),
                                      jnp.zeros((16,), jnp.float32))
                return (s2, c2c)

            return lax.fori_loop(0, CHUNK_S // 16, grp, (ssum, scnt))

        init = (jnp.zeros((16,), jnp.float32), jnp.zeros((16,), jnp.float32))
        return lax.fori_loop(0, nchunks, chunk_body, init)

    ppw = E // NW
    npw = NNEG_PAD // NW
    ps, pc = make_loop(ppw // CHUNK_S, w * ppw, row_hbm, col_hbm, True)
    qs, qc = make_loop(npw // CHUNK_S, w * npw, nr_hbm, nc_hbm, False)
    res_v[pl.ds(0, 16)] = ps
    res_v[pl.ds(16, 16)] = pc
    res_v[pl.ds(32, 16)] = qs
    res_v[pl.ds(48, 16)] = qc
    pltpu.sync_copy(res_v, out_hbm.at[w])


def _sc_score(rep_pad, row, col, nr_pad, nc_pad):
    k = functools.partial(
        pl.kernel,
        out_type=jax.ShapeDtypeStruct((NW, 4 * 16), jnp.float32),
        mesh=_mesh,
        compiler_params=_sc_params_nl,
        scratch_types=[
            pltpu.VMEM_SHARED((NPAD, H), jnp.float32),
            pltpu.VMEM((CHUNK_S,), jnp.int32),
            pltpu.VMEM((CHUNK_S,), jnp.int32),
            pltpu.VMEM((CHUNK_S, H), jnp.float32),
            pltpu.VMEM((CHUNK_S, H), jnp.float32),
            pltpu.VMEM((4 * 16,), jnp.float32),
        ],
    )(_sc_score_body)
    return k(rep_pad, row, col, nr_pad, nc_pad)


def _mm1_body(x_ref, w_ref, o_ref):
    o_ref[...] = jnp.dot(x_ref[...], w_ref[...],
                         preferred_element_type=jnp.float32)


def _tc_mm1(features, W1):
    return pl.pallas_call(
        _mm1_body,
        out_shape=jax.ShapeDtypeStruct((N, H), jnp.float32),
    )(features, W1)


def _prep_body(degt_ref, xw_ref, dinv_ref, y_ref):
    d = degt_ref[...]
    deg = d[:, 0:1] + d[:, 1:2] + 1.0
    dinv = lax.rsqrt(deg)
    dinv_ref[...] = dinv
    y_ref[0:N, :] = dinv[0:N] * xw_ref[...]
    y_ref[N:NPAD, :] = jnp.zeros((NPAD - N, H), jnp.float32)


def _tc_prep(degt, xw1):
    return pl.pallas_call(
        _prep_body,
        out_shape=[jax.ShapeDtypeStruct((NPAD, 1), jnp.float32),
                   jax.ShapeDtypeStruct((NPAD, H), jnp.float32)],
    )(degt, xw1)


def _mid_body(acc_ref, y_ref, dinv_ref, b1_ref, w2_ref, o_ref):
    dinv = dinv_ref[...]
    s = acc_ref[0] + acc_ref[1] + y_ref[...]
    h = jnp.maximum(dinv * s + b1_ref[...], 0.0)
    xw2 = jnp.dot(h, w2_ref[...], preferred_element_type=jnp.float32)
    y2 = dinv * xw2
    o_ref[0:N, :] = y2[0:N]
    o_ref[N:NPAD, :] = jnp.zeros((NPAD - N, H), jnp.float32)


def _tc_mid(accp1, y1p, dinvp, b1, W2):
    return pl.pallas_call(
        _mid_body,
        out_shape=jax.ShapeDtypeStruct((NPAD, H), jnp.float32),
    )(accp1, y1p, dinvp, b1, W2)


def _final_body(acc_ref, y_ref, dinv_ref, b2_ref, o_ref):
    s = acc_ref[0, 0:N] + acc_ref[1, 0:N] + y_ref[0:N]
    o_ref[...] = dinv_ref[0:N] * s + b2_ref[...]


def _tc_final(accp2, y2p, dinvp, b2):
    return pl.pallas_call(
        _final_body,
        out_shape=jax.ShapeDtypeStruct((N, H), jnp.float32),
    )(accp2, y2p, dinvp, b2)


def _loss_body(parts_ref, o_ref):
    p = parts_ref[...]
    pos_sq = jnp.sum(p[:, 0:16])
    pos_c = jnp.sum(p[:, 16:32])
    neg_sq = jnp.sum(p[:, 32:48])
    neg_c = jnp.sum(p[:, 48:64])
    rec = (neg_sq + pos_sq) * jnp.float32(N) / (neg_c + pos_c)
    o_ref[...] = jnp.broadcast_to(rec, (1, 1))


def _tc_loss(parts):
    return pl.pallas_call(
        _loss_body,
        out_shape=jax.ShapeDtypeStruct((1, 1), jnp.float32),
    )(parts)


def kernel(features, edge_index, neg_edge_index, W1, b1, W2, b2):
    assert features.shape == (N, F_IN)
    assert edge_index.shape == (2, E)
    assert neg_edge_index.shape == (2, NNEG)

    row = edge_index[0]
    col = edge_index[1]
    nr = neg_edge_index[0]
    nc = neg_edge_index[1]
    zpad = jnp.zeros((NNEG_PAD - NNEG,), jnp.int32)
    nr_p = jnp.concatenate([nr, zpad])
    nc_p = jnp.concatenate([nc, zpad])

    xw1 = _tc_mm1(features, W1)
    degp = _sc_degree(col)
    degt = jnp.transpose(degp)
    dinvp, y1p = _tc_prep(degt, xw1)
    accp1 = _sc_edge_pass(y1p, row, col)
    y2p = _tc_mid(accp1, y1p, dinvp, b1, W2)
    accp2 = _sc_edge_pass(y2p, row, col)
    rep = _tc_final(accp2, y2p, dinvp, b2)
    rep_pad = jnp.concatenate(
        [rep, jnp.zeros((NPAD - N, H), jnp.float32)], axis=0)
    parts = _sc_score(rep_pad, row, col, nr_p, nc_p)
    rec_loss = _tc_loss(parts)[0, 0]
    return (rep, rec_loss)


# speedup vs baseline: 30.9869x; 1.5725x over previous
import functools

import jax
import jax.numpy as jnp
from jax import lax
from jax.experimental import pallas as pl
from jax.experimental.pallas import tpu as pltpu
from jax.experimental.pallas import tpu_sc as plsc

N = 10000
NPAD = 10240
E = 320000
NNEG = 50000
NNEG_PAD = 51200
F_IN = 128
H = 64
NC = 2
NS = 16
NW = NC * NS
RPT = NPAD // NS
FH = H // NC

CHUNK_D = 2000
CK = 400
NCH_P = E // CK
NCH_N = NNEG_PAD // CK
CPT_P = NCH_P // NS
CPT_N = NCH_N // NS
ETOT = E + NNEG_PAD

_mesh = plsc.VectorSubcoreMesh(core_axis_name="c", subcore_axis_name="s")
_sc_params = pltpu.CompilerParams(use_tc_tiling_on_sc=False)
_sc_params_nl = pltpu.CompilerParams(use_tc_tiling_on_sc=False,
                                     needs_layout_passes=False)


def _sc_degree_body(col_hbm, out_hbm, deg_sp, cidx_v, ones_v):
    cid = lax.axis_index("c")
    sid = lax.axis_index("s")

    def zfill(i, _):
        ones_v[pl.ds(i * 16, 16)] = jnp.zeros((16,), jnp.float32)
        return 0

    lax.fori_loop(0, RPT // 16, zfill, 0)
    pltpu.sync_copy(ones_v.at[pl.ds(0, RPT)], deg_sp.at[pl.ds(sid * RPT, RPT)])

    def fill(i, _):
        ones_v[pl.ds(i * 16, 16)] = jnp.ones((16,), jnp.float32)
        return 0

    lax.fori_loop(0, CHUNK_D // 16, fill, 0)
    plsc.subcore_barrier()

    epw = E // NW
    base = (cid * NS + sid) * epw

    def body(k, _):
        pltpu.sync_copy(col_hbm.at[pl.ds(base + k * CHUNK_D, CHUNK_D)], cidx_v)
        pltpu.sync_copy(ones_v, deg_sp.at[cidx_v], add=True)
        return 0

    lax.fori_loop(0, epw // CHUNK_D, body, 0)
    plsc.subcore_barrier()
    pltpu.sync_copy(deg_sp.at[pl.ds(sid * RPT, RPT)],
                    out_hbm.at[cid, pl.ds(sid * RPT, RPT)])


def _sc_degree(col):
    k = functools.partial(
        pl.kernel,
        out_type=jax.ShapeDtypeStruct((NC, NPAD), jnp.float32),
        mesh=_mesh,
        compiler_params=_sc_params,
        scratch_types=[
            pltpu.VMEM_SHARED((NPAD,), jnp.float32),
            pltpu.VMEM((CHUNK_D,), jnp.int32),
            pltpu.VMEM((CHUNK_D,), jnp.float32),
        ],
    )(_sc_degree_body)
    return k(col)


def _sc_edge_body(y_hbm, row2d_hbm, col2d_hbm, out_hbm,
                  y_sp, acc_sp, ridx_all, cidx_all,
                  rows_a, rows_b, sem_a, sem_b):
    cid = lax.axis_index("c")
    sid = lax.axis_index("s")
    fbase = cid * FH
    r0 = sid * RPT

    def zfill(i, _):
        rows_a[i // 2, pl.ds((i % 2) * 16, 16)] = jnp.zeros((16,), jnp.float32)
        return 0

    lax.fori_loop(0, CK * 2, zfill, 0)
    pltpu.sync_copy(rows_a, acc_sp.at[pl.ds(r0, CK)])
    pltpu.sync_copy(rows_a.at[pl.ds(0, RPT - CK)],
                    acc_sp.at[pl.ds(r0 + CK, RPT - CK)])
    pltpu.sync_copy(y_hbm.at[pl.ds(r0, RPT), pl.ds(fbase, FH)],
                    y_sp.at[pl.ds(r0, RPT)])
    pltpu.sync_copy(row2d_hbm.at[pl.ds(sid * CPT_P, CPT_P)], ridx_all)
    pltpu.sync_copy(col2d_hbm.at[pl.ds(sid * CPT_P, CPT_P)], cidx_all)
    plsc.subcore_barrier()

    pltpu.async_copy(y_sp.at[ridx_all.at[0]], rows_a, sem_a)

    def body(j, _):
        k0 = 2 * j
        pltpu.make_async_copy(y_sp.at[ridx_all.at[k0]], rows_a, sem_a).wait()
        pltpu.async_copy(y_sp.at[ridx_all.at[k0 + 1]], rows_b, sem_b)
        pltpu.sync_copy(rows_a, acc_sp.at[cidx_all.at[k0]], add=True)
        pltpu.make_async_copy(y_sp.at[ridx_all.at[k0 + 1]], rows_b,
                              sem_b).wait()

        @pl.when(k0 + 2 < CPT_P)
        def _():
            pltpu.async_copy(y_sp.at[ridx_all.at[k0 + 2]], rows_a, sem_a)

        pltpu.sync_copy(rows_b, acc_sp.at[cidx_all.at[k0 + 1]], add=True)
        return 0

    lax.fori_loop(0, CPT_P // 2, body, 0)
    plsc.subcore_barrier()
    pltpu.sync_copy(acc_sp.at[pl.ds(r0, RPT)],
                    out_hbm.at[pl.ds(r0, RPT), pl.ds(fbase, FH)])


def _sc_edge_pass(y_pad, row2d, col2d):
    k = functools.partial(
        pl.kernel,
        out_type=jax.ShapeDtypeStruct((NPAD, H), jnp.float32),
        mesh=_mesh,
        compiler_params=_sc_params,
        scratch_types=[
            pltpu.VMEM_SHARED((NPAD, FH), jnp.float32),
            pltpu.VMEM_SHARED((NPAD, FH), jnp.float32),
            pltpu.VMEM((CPT_P, CK), jnp.int32),
            pltpu.VMEM((CPT_P, CK), jnp.int32),
            pltpu.VMEM((CK, FH), jnp.float32),
            pltpu.VMEM((CK, FH), jnp.float32),
            pltpu.SemaphoreType.DMA,
            pltpu.SemaphoreType.DMA,
        ],
    )(_sc_edge_body)
    return k(y_pad, row2d, col2d)


def _sc_score_body(rep_hbm, pr_hbm, pc_hbm, nr_hbm, nc_hbm, out_hbm,
                   rep_sp, pr_idx, pc_idx, nr_idx, nc_idx,
                   ar0, br0, ar1, br1, d0, d1,
                   sem_g0, sem_g1, sem_w0, sem_w1):
    cid = lax.axis_index("c")
    sid = lax.axis_index("s")
    fbase = cid * FH
    r0 = sid * RPT

    pltpu.sync_copy(rep_hbm.at[pl.ds(r0, RPT), pl.ds(fbase, FH)],
                    rep_sp.at[pl.ds(r0, RPT)])
    pltpu.sync_copy(pr_hbm.at[pl.ds(sid * CPT_P, CPT_P)], pr_idx)
    pltpu.sync_copy(pc_hbm.at[pl.ds(sid * CPT_P, CPT_P)], pc_idx)
    pltpu.sync_copy(nr_hbm.at[pl.ds(sid * CPT_N, CPT_N)], nr_idx)
    pltpu.sync_copy(nc_hbm.at[pl.ds(sid * CPT_N, CPT_N)], nc_idx)
    plsc.subcore_barrier()

    lane = lax.iota(jnp.int32, 16)

    def compute(arows, brows, dbuf):
        def grp(g, _):
            rows0 = g * 16 + lane
            accs = [jnp.zeros((16,), jnp.float32) for _ in range(4)]
            for f in range(FH):
                colsf = jnp.bitwise_and(f + lane, FH - 1)
                a = plsc.load_gather(arows, [rows0, colsf])
                b = plsc.load_gather(brows, [rows0, colsf])
                accs[f % 4] = accs[f % 4] + a * b
            dbuf[pl.ds(g * 16, 16)] = (accs[0] + accs[1]) + (accs[2] + accs[3])
            return 0

        lax.fori_loop(0, CK // 16, grp, 0)

    def run(cpt, ridx, cidx, obase):
        def gather(t, ar, br, sem):
            pltpu.async_copy(rep_sp.at[ridx.at[t]], ar, sem)
            pltpu.async_copy(rep_sp.at[cidx.at[t]], br, sem)

        def drain(t, ar, br, sem):
            pltpu.make_async_copy(rep_sp.at[ridx.at[t]], ar, sem).wait()
            pltpu.make_async_copy(rep_sp.at[cidx.at[t]], br, sem).wait()

        gather(0, ar0, br0, sem_g0)

        def body(j, _):
            k0 = 2 * j
            drain(k0, ar0, br0, sem_g0)
            gather(k0 + 1, ar1, br1, sem_g1)

            @pl.when(j > 0)
            def _():
                pltpu.make_async_copy(
                    d0, out_hbm.at[cid, pl.ds(0, CK)], sem_w0).wait()

            compute(ar0, br0, d0)
            off0 = obase + (sid * cpt + k0) * CK
            pltpu.async_copy(d0, out_hbm.at[cid, pl.ds(off0, CK)], sem_w0)

            drain(k0 + 1, ar1, br1, sem_g1)

            @pl.when(k0 + 2 < cpt)
            def _():
                gather(k0 + 2, ar0, br0, sem_g0)

            @pl.when(j > 0)
            def _():
                pltpu.make_async_copy(
                    d1, out_hbm.at[cid, pl.ds(0, CK)], sem_w1).wait()

            compute(ar1, br1, d1)
            off1 = off0 + CK
            pltpu.async_copy(d1, out_hbm.at[cid, pl.ds(off1, CK)], sem_w1)
            return 0

        lax.fori_loop(0, cpt // 2, body, 0)
        pltpu.make_async_copy(d0, out_hbm.at[cid, pl.ds(0, CK)], sem_w0).wait()
        pltpu.make_async_copy(d1, out_hbm.at[cid, pl.ds(0, CK)], sem_w1).wait()

    run(CPT_P, pr_idx, pc_idx, 0)
    run(CPT_N, nr_idx, nc_idx, E)


def _sc_score(rep_pad, pr2d, pc2d, nr2d, nc2d):
    k = functools.partial(
        pl.kernel,
        out_type=jax.ShapeDtypeStruct((NC, ETOT), jnp.float32),
        mesh=_mesh,
        compiler_params=_sc_params_nl,
        scratch_types=[
            pltpu.VMEM_SHARED((NPAD, FH), jnp.float32),
            pltpu.VMEM((CPT_P, CK), jnp.int32),
            pltpu.VMEM((CPT_P, CK), jnp.int32),
            pltpu.VMEM((CPT_N, CK), jnp.int32),
            pltpu.VMEM((CPT_N, CK), jnp.int32),
            pltpu.VMEM((CK, FH), jnp.float32),
            pltpu.VMEM((CK, FH), jnp.float32),
            pltpu.VMEM((CK, FH), jnp.float32),
            pltpu.VMEM((CK, FH), jnp.float32),
            pltpu.VMEM((CK,), jnp.float32),
            pltpu.VMEM((CK,), jnp.float32),
            pltpu.SemaphoreType.DMA,
            pltpu.SemaphoreType.DMA,
            pltpu.SemaphoreType.DMA,
            pltpu.SemaphoreType.DMA,
        ],
    )(_sc_score_body)
    return k(rep_pad, pr2d, pc2d, nr2d, nc2d)


def _mm1_body(x_ref, w_ref, o_ref):
    o_ref[...] = jnp.dot(x_ref[...], w_ref[...],
                         preferred_element_type=jnp.float32)


def _tc_mm1(features, W1):
    return pl.pallas_call(
        _mm1_body,
        out_shape=jax.ShapeDtypeStruct((N, H), jnp.float32),
    )(features, W1)


def _prep_body(degt_ref, xw_ref, dinv_ref, y_ref):
    d = degt_ref[...]
    deg = d[:, 0:1] + d[:, 1:2] + 1.0
    dinv = lax.rsqrt(deg)
    dinv_ref[...] = dinv
    y_ref[0:N, :] = dinv[0:N] * xw_ref[...]
    y_ref[N:NPAD, :] = jnp.zeros((NPAD - N, H), jnp.float32)


def _tc_prep(degt, xw1):
    return pl.pallas_call(
        _prep_body,
        out_shape=[jax.ShapeDtypeStruct((NPAD, 1), jnp.float32),
                   jax.ShapeDtypeStruct((NPAD, H), jnp.float32)],
    )(degt, xw1)


def _mid_body(acc_ref, y_ref, dinv_ref, b1_ref, w2_ref, o_ref):
    dinv = dinv_ref[...]
    s = acc_ref[...] + y_ref[...]
    h = jnp.maximum(dinv * s + b1_ref[...], 0.0)
    xw2 = jnp.dot(h, w2_ref[...], preferred_element_type=jnp.float32)
    y2 = dinv * xw2
    o_ref[0:N, :] = y2[0:N]
    o_ref[N:NPAD, :] = jnp.zeros((NPAD - N, H), jnp.float32)


def _tc_mid(acc1, y1p, dinvp, b1, W2):
    return pl.pallas_call(
        _mid_body,
        out_shape=jax.ShapeDtypeStruct((NPAD, H), jnp.float32),
    )(acc1, y1p, dinvp, b1, W2)


def _final_body(acc_ref, y_ref, dinv_ref, b2_ref, o_ref):
    s = acc_ref[0:N] + y_ref[0:N]
    o_ref[...] = dinv_ref[0:N] * s + b2_ref[...]


def _tc_final(acc2, y2p, dinvp, b2):
    return pl.pallas_call(
        _final_body,
        out_shape=jax.ShapeDtypeStruct((N, H), jnp.float32),
    )(acc2, y2p, dinvp, b2)


def _combine_body(dots_ref, pr_ref, pc_ref, nr_ref, nc_ref, o_ref):
    dp = dots_ref[0] + dots_ref[1]
    pos_d = dp[0:E // 128]
    neg_d = dp[E // 128:ETOT // 128]
    mp = (pr_ref[...] < pc_ref[...]).astype(jnp.float32)
    mn = (nr_ref[...] < nc_ref[...]).astype(jnp.float32)
    t = pos_d - 1.0
    s_pos = jnp.sum(mp * t * t)
    s_neg = jnp.sum(mn * neg_d * neg_d)
    denom = jnp.sum(mp) + jnp.sum(mn)
    rec = (s_neg + s_pos) * jnp.float32(N) / denom
    o_ref[...] = jnp.broadcast_to(rec, (1, 1))


def _tc_combine(dots3d, pr, pc, nr, nc):
    return pl.pallas_call(
        _combine_body,
        out_shape=jax.ShapeDtypeStruct((1, 1), jnp.float32),
    )(dots3d, pr, pc, nr, nc)


def kernel(features, edge_index, neg_edge_index, W1, b1, W2, b2):
    assert features.shape == (N, F_IN)
    assert edge_index.shape == (2, E)
    assert neg_edge_index.shape == (2, NNEG)

    row = edge_index[0]
    col = edge_index[1]
    nr = neg_edge_index[0]
    nc = neg_edge_index[1]
    zpad = jnp.zeros((NNEG_PAD - NNEG,), jnp.int32)
    nr_p = jnp.concatenate([nr, zpad])
    nc_p = jnp.concatenate([nc, zpad])
    row2d = row.reshape(NCH_P, CK)
    col2d = col.reshape(NCH_P, CK)
    nr2d = nr_p.reshape(NCH_N, CK)
    nc2d = nc_p.reshape(NCH_N, CK)

    xw1 = _tc_mm1(features, W1)
    degp = _sc_degree(col)
    degt = jnp.transpose(degp)
    dinvp, y1p = _tc_prep(degt, xw1)
    acc1 = _sc_edge_pass(y1p, row2d, col2d)
    y2p = _tc_mid(acc1, y1p, dinvp, b1, W2)
    acc2 = _sc_edge_pass(y2p, row2d, col2d)
    rep = _tc_final(acc2, y2p, dinvp, b2)
    rep_pad = jnp.concatenate(
        [rep, jnp.zeros((NPAD - N, H), jnp.float32)], axis=0)
    dots = _sc_score(rep_pad, row2d, col2d, nr2d, nc2d)
    dots3d = dots.reshape(NC, ETOT // 128, 128)
    rec_loss = _tc_combine(dots3d,
                           row.reshape(E // 128, 128),
                           col.reshape(E // 128, 128),
                           nr_p.reshape(NNEG_PAD // 128, 128),
                           nc_p.reshape(NNEG_PAD // 128, 128))[0, 0]
    return (rep, rec_loss)


# speedup vs baseline: 31.3339x; 1.0112x over previous
import functools

import jax
import jax.numpy as jnp
from jax import lax
from jax.experimental import pallas as pl
from jax.experimental.pallas import tpu as pltpu
from jax.experimental.pallas import tpu_sc as plsc

N = 10000
NPAD = 10240
E = 320000
NNEG = 50000
NNEG_PAD = 51200
F_IN = 128
H = 64
NC = 2
NS = 16
NW = NC * NS
RPT = NPAD // NS
FH = H // NC

CHUNK_D = 2000
CK = 400
NCH_P = E // CK
NCH_N = NNEG_PAD // CK
CPT_P = NCH_P // NS
CPT_N = NCH_N // NS
ETOT = E + NNEG_PAD

_mesh = plsc.VectorSubcoreMesh(core_axis_name="c", subcore_axis_name="s")
_sc_params = pltpu.CompilerParams(use_tc_tiling_on_sc=False)
_sc_params_nl = pltpu.CompilerParams(use_tc_tiling_on_sc=False,
                                     needs_layout_passes=False)


def _sc_degree_body(col_hbm, out_hbm, deg_sp, cidx_v, ones_v):
    cid = lax.axis_index("c")
    sid = lax.axis_index("s")

    def zfill(i, _):
        ones_v[pl.ds(i * 16, 16)] = jnp.zeros((16,), jnp.float32)
        return 0

    lax.fori_loop(0, RPT // 16, zfill, 0)
    pltpu.sync_copy(ones_v.at[pl.ds(0, RPT)], deg_sp.at[pl.ds(sid * RPT, RPT)])

    def fill(i, _):
        ones_v[pl.ds(i * 16, 16)] = jnp.ones((16,), jnp.float32)
        return 0

    lax.fori_loop(0, CHUNK_D // 16, fill, 0)
    plsc.subcore_barrier()

    epw = E // NW
    base = (cid * NS + sid) * epw

    def body(k, _):
        pltpu.sync_copy(col_hbm.at[pl.ds(base + k * CHUNK_D, CHUNK_D)], cidx_v)
        pltpu.sync_copy(ones_v, deg_sp.at[cidx_v], add=True)
        return 0

    lax.fori_loop(0, epw // CHUNK_D, body, 0)
    plsc.subcore_barrier()
    pltpu.sync_copy(deg_sp.at[pl.ds(sid * RPT, RPT)],
                    out_hbm.at[cid, pl.ds(sid * RPT, RPT)])


def _sc_degree(col):
    k = functools.partial(
        pl.kernel,
        out_type=jax.ShapeDtypeStruct((NC, NPAD), jnp.float32),
        mesh=_mesh,
        compiler_params=_sc_params,
        scratch_types=[
            pltpu.VMEM_SHARED((NPAD,), jnp.float32),
            pltpu.VMEM((CHUNK_D,), jnp.int32),
            pltpu.VMEM((CHUNK_D,), jnp.float32),
        ],
    )(_sc_degree_body)
    return k(col)


def _sc_edge_body(y_hbm, row2d_hbm, col2d_hbm, out_hbm,
                  y_sp, acc_sp, ridx_all, cidx_all,
                  rows_a, rows_b, sem_a, sem_b):
    cid = lax.axis_index("c")
    sid = lax.axis_index("s")
    fbase = cid * FH
    r0 = sid * RPT

    def zfill(i, _):
        rows_a[i // 2, pl.ds((i % 2) * 16, 16)] = jnp.zeros((16,), jnp.float32)
        return 0

    lax.fori_loop(0, CK * 2, zfill, 0)
    pltpu.sync_copy(rows_a, acc_sp.at[pl.ds(r0, CK)])
    pltpu.sync_copy(rows_a.at[pl.ds(0, RPT - CK)],
                    acc_sp.at[pl.ds(r0 + CK, RPT - CK)])
    pltpu.sync_copy(y_hbm.at[pl.ds(r0, RPT), pl.ds(fbase, FH)],
                    y_sp.at[pl.ds(r0, RPT)])
    pltpu.sync_copy(row2d_hbm.at[pl.ds(sid * CPT_P, CPT_P)], ridx_all)
    pltpu.sync_copy(col2d_hbm.at[pl.ds(sid * CPT_P, CPT_P)], cidx_all)
    plsc.subcore_barrier()

    pltpu.async_copy(y_sp.at[ridx_all.at[0]], rows_a, sem_a)

    def body(j, _):
        k0 = 2 * j
        pltpu.make_async_copy(y_sp.at[ridx_all.at[k0]], rows_a, sem_a).wait()
        pltpu.async_copy(y_sp.at[ridx_all.at[k0 + 1]], rows_b, sem_b)
        pltpu.sync_copy(rows_a, acc_sp.at[cidx_all.at[k0]], add=True)
        pltpu.make_async_copy(y_sp.at[ridx_all.at[k0 + 1]], rows_b,
                              sem_b).wait()

        @pl.when(k0 + 2 < CPT_P)
        def _():
            pltpu.async_copy(y_sp.at[ridx_all.at[k0 + 2]], rows_a, sem_a)

        pltpu.sync_copy(rows_b, acc_sp.at[cidx_all.at[k0 + 1]], add=True)
        return 0

    lax.fori_loop(0, CPT_P // 2, body, 0)
    plsc.subcore_barrier()
    pltpu.sync_copy(acc_sp.at[pl.ds(r0, RPT)],
                    out_hbm.at[pl.ds(r0, RPT), pl.ds(fbase, FH)])


def _sc_edge_pass(y_pad, row2d, col2d):
    k = functools.partial(
        pl.kernel,
        out_type=jax.ShapeDtypeStruct((NPAD, H), jnp.float32),
        mesh=_mesh,
        compiler_params=_sc_params,
        scratch_types=[
            pltpu.VMEM_SHARED((NPAD, FH), jnp.float32),
            pltpu.VMEM_SHARED((NPAD, FH), jnp.float32),
            pltpu.VMEM((CPT_P, CK), jnp.int32),
            pltpu.VMEM((CPT_P, CK), jnp.int32),
            pltpu.VMEM((CK, FH), jnp.float32),
            pltpu.VMEM((CK, FH), jnp.float32),
            pltpu.SemaphoreType.DMA,
            pltpu.SemaphoreType.DMA,
        ],
    )(_sc_edge_body)
    return k(y_pad, row2d, col2d)


def _sc_score_body(rep_hbm, pr_hbm, pc_hbm, nr_hbm, nc_hbm, out_hbm,
                   rep_sp, pr_idx, pc_idx, nr_idx, nc_idx,
                   ar0, br0, ar1, br1, d0, d1,
                   sem_g0, sem_g1, sem_w0, sem_w1):
    cid = lax.axis_index("c")
    sid = lax.axis_index("s")
    fbase = cid * FH
    r0 = sid * RPT

    pltpu.sync_copy(rep_hbm.at[pl.ds(r0, RPT), pl.ds(fbase, FH)],
                    rep_sp.at[pl.ds(r0, RPT)])
    pltpu.sync_copy(pr_hbm.at[pl.ds(sid * CPT_P, CPT_P)], pr_idx)
    pltpu.sync_copy(pc_hbm.at[pl.ds(sid * CPT_P, CPT_P)], pc_idx)
    pltpu.sync_copy(nr_hbm.at[pl.ds(sid * CPT_N, CPT_N)], nr_idx)
    pltpu.sync_copy(nc_hbm.at[pl.ds(sid * CPT_N, CPT_N)], nc_idx)
    plsc.subcore_barrier()

    lane = lax.iota(jnp.int32, 16)

    def compute(arows, brows, dbuf):
        def grp(g, _):
            rows0 = g * 16 + lane
            accs = [jnp.zeros((16,), jnp.float32) for _ in range(4)]
            for f in range(FH):
                colsf = jnp.bitwise_and(f + lane, FH - 1)
                a = plsc.load_gather(arows, [rows0, colsf])
                b = plsc.load_gather(brows, [rows0, colsf])
                accs[f % 4] = accs[f % 4] + a * b
            dbuf[pl.ds(g * 16, 16)] = (accs[0] + accs[1]) + (accs[2] + accs[3])
            return 0

        lax.fori_loop(0, CK // 16, grp, 0)

    def run(cpt, ridx, cidx, obase):
        def gather(t, ar, br, sem):
            pltpu.async_copy(rep_sp.at[ridx.at[t]], ar, sem)
            pltpu.async_copy(rep_sp.at[cidx.at[t]], br, sem)

        def drain(t, ar, br, sem):
            pltpu.make_async_copy(rep_sp.at[ridx.at[t]], ar, sem).wait()
            pltpu.make_async_copy(rep_sp.at[cidx.at[t]], br, sem).wait()

        gather(0, ar0, br0, sem_g0)

        def body(j, _):
            k0 = 2 * j
            drain(k0, ar0, br0, sem_g0)
            gather(k0 + 1, ar1, br1, sem_g1)

            @pl.when(j > 0)
            def _():
                pltpu.make_async_copy(
                    d0, out_hbm.at[cid, pl.ds(0, CK)], sem_w0).wait()

            compute(ar0, br0, d0)
            off0 = obase + (sid * cpt + k0) * CK
            pltpu.async_copy(d0, out_hbm.at[cid, pl.ds(off0, CK)], sem_w0)

            drain(k0 + 1, ar1, br1, sem_g1)

            @pl.when(k0 + 2 < cpt)
            def _():
                gather(k0 + 2, ar0, br0, sem_g0)

            @pl.when(j > 0)
            def _():
                pltpu.make_async_copy(
                    d1, out_hbm.at[cid, pl.ds(0, CK)], sem_w1).wait()

            compute(ar1, br1, d1)
            off1 = off0 + CK
            pltpu.async_copy(d1, out_hbm.at[cid, pl.ds(off1, CK)], sem_w1)
            return 0

        lax.fori_loop(0, cpt // 2, body, 0)
        pltpu.make_async_copy(d0, out_hbm.at[cid, pl.ds(0, CK)], sem_w0).wait()
        pltpu.make_async_copy(d1, out_hbm.at[cid, pl.ds(0, CK)], sem_w1).wait()

    run(CPT_P, pr_idx, pc_idx, 0)
    run(CPT_N, nr_idx, nc_idx, E)


def _sc_score(rep_pad, pr2d, pc2d, nr2d, nc2d):
    k = functools.partial(
        pl.kernel,
        out_type=jax.ShapeDtypeStruct((NC, ETOT), jnp.float32),
        mesh=_mesh,
        compiler_params=_sc_params_nl,
        scratch_types=[
            pltpu.VMEM_SHARED((NPAD, FH), jnp.float32),
            pltpu.VMEM((CPT_P, CK), jnp.int32),
            pltpu.VMEM((CPT_P, CK), jnp.int32),
            pltpu.VMEM((CPT_N, CK), jnp.int32),
            pltpu.VMEM((CPT_N, CK), jnp.int32),
            pltpu.VMEM((CK, FH), jnp.float32),
            pltpu.VMEM((CK, FH), jnp.float32),
            pltpu.VMEM((CK, FH), jnp.float32),
            pltpu.VMEM((CK, FH), jnp.float32),
            pltpu.VMEM((CK,), jnp.float32),
            pltpu.VMEM((CK,), jnp.float32),
            pltpu.SemaphoreType.DMA,
            pltpu.SemaphoreType.DMA,
            pltpu.SemaphoreType.DMA,
            pltpu.SemaphoreType.DMA,
        ],
    )(_sc_score_body)
    return k(rep_pad, pr2d, pc2d, nr2d, nc2d)


def _prep_body(x_ref, w_ref, degt_ref, dinv_ref, y_ref):
    d = degt_ref[...]
    deg = d[:, 0:1] + d[:, 1:2] + 1.0
    dinv = lax.rsqrt(deg)
    dinv_ref[...] = dinv
    xw = jnp.dot(x_ref[...], w_ref[...], preferred_element_type=jnp.float32)
    y_ref[0:N, :] = dinv[0:N] * xw
    y_ref[N:NPAD, :] = jnp.zeros((NPAD - N, H), jnp.float32)


def _tc_prep(features, W1, degt):
    return pl.pallas_call(
        _prep_body,
        out_shape=[jax.ShapeDtypeStruct((NPAD, 1), jnp.float32),
                   jax.ShapeDtypeStruct((NPAD, H), jnp.float32)],
    )(features, W1, degt)


def _mid_body(acc_ref, y_ref, dinv_ref, b1_ref, w2_ref, o_ref):
    dinv = dinv_ref[...]
    s = acc_ref[...] + y_ref[...]
    h = jnp.maximum(dinv * s + b1_ref[...], 0.0)
    xw2 = jnp.dot(h, w2_ref[...], preferred_element_type=jnp.float32)
    y2 = dinv * xw2
    o_ref[0:N, :] = y2[0:N]
    o_ref[N:NPAD, :] = jnp.zeros((NPAD - N, H), jnp.float32)


def _tc_mid(acc1, y1p, dinvp, b1, W2):
    return pl.pallas_call(
        _mid_body,
        out_shape=jax.ShapeDtypeStruct((NPAD, H), jnp.float32),
    )(acc1, y1p, dinvp, b1, W2)


def _final_body(acc_ref, y_ref, dinv_ref, b2_ref, o_ref, p_ref):
    s = acc_ref[0:N] + y_ref[0:N]
    rep = dinv_ref[0:N] * s + b2_ref[...]
    o_ref[...] = rep
    p_ref[0:N, :] = rep
    p_ref[N:NPAD, :] = jnp.zeros((NPAD - N, H), jnp.float32)


def _tc_final(acc2, y2p, dinvp, b2):
    return pl.pallas_call(
        _final_body,
        out_shape=[jax.ShapeDtypeStruct((N, H), jnp.float32),
                   jax.ShapeDtypeStruct((NPAD, H), jnp.float32)],
    )(acc2, y2p, dinvp, b2)


def _combine_body(dots_ref, pr_ref, pc_ref, nr_ref, nc_ref, o_ref):
    dp = dots_ref[0] + dots_ref[1]
    pos_d = dp[0:E // 128]
    neg_d = dp[E // 128:ETOT // 128]
    mp = (pr_ref[...] < pc_ref[...]).astype(jnp.float32)
    mn = (nr_ref[...] < nc_ref[...]).astype(jnp.float32)
    t = pos_d - 1.0
    s_pos = jnp.sum(mp * t * t)
    s_neg = jnp.sum(mn * neg_d * neg_d)
    denom = jnp.sum(mp) + jnp.sum(mn)
    rec = (s_neg + s_pos) * jnp.float32(N) / denom
    o_ref[...] = jnp.broadcast_to(rec, (1, 1))


def _tc_combine(dots3d, pr, pc, nr, nc):
    return pl.pallas_call(
        _combine_body,
        out_shape=jax.ShapeDtypeStruct((1, 1), jnp.float32),
    )(dots3d, pr, pc, nr, nc)


def kernel(features, edge_index, neg_edge_index, W1, b1, W2, b2):
    assert features.shape == (N, F_IN)
    assert edge_index.shape == (2, E)
    assert neg_edge_index.shape == (2, NNEG)

    row = edge_index[0]
    col = edge_index[1]
    nr = neg_edge_index[0]
    nc = neg_edge_index[1]
    zpad = jnp.zeros((NNEG_PAD - NNEG,), jnp.int32)
    nr_p = jnp.concatenate([nr, zpad])
    nc_p = jnp.concatenate([nc, zpad])
    row2d = row.reshape(NCH_P, CK)
    col2d = col.reshape(NCH_P, CK)
    nr2d = nr_p.reshape(NCH_N, CK)
    nc2d = nc_p.reshape(NCH_N, CK)

    degp = _sc_degree(col)
    degt = jnp.transpose(degp)
    dinvp, y1p = _tc_prep(features, W1, degt)
    acc1 = _sc_edge_pass(y1p, row2d, col2d)
    y2p = _tc_mid(acc1, y1p, dinvp, b1, W2)
    acc2 = _sc_edge_pass(y2p, row2d, col2d)
    rep, rep_pad = _tc_final(acc2, y2p, dinvp, b2)
    dots = _sc_score(rep_pad, row2d, col2d, nr2d, nc2d)
    dots3d = dots.reshape(NC, ETOT // 128, 128)
    rec_loss = _tc_combine(dots3d,
                           row.reshape(E // 128, 128),
                           col.reshape(E // 128, 128),
                           nr_p.reshape(NNEG_PAD // 128, 128),
                           nc_p.reshape(NNEG_PAD // 128, 128))[0, 0]
    return (rep, rec_loss)


# speedup vs baseline: 32.4194x; 1.0346x over previous
import functools

import jax
import jax.numpy as jnp
from jax import lax
from jax.experimental import pallas as pl
from jax.experimental.pallas import tpu as pltpu
from jax.experimental.pallas import tpu_sc as plsc

N = 10000
NPAD = 10240
E = 320000
NNEG = 50000
NNEG_PAD = 51200
F_IN = 128
H = 64
NC = 2
NS = 16
NW = NC * NS
RPT = NPAD // NS
FH = H // NC

CHUNK_D = 2000
CK = 400
NCH_P = E // CK
NCH_N = NNEG_PAD // CK
CPT_P = NCH_P // NS
CPT_N = NCH_N // NS
ETOT = E + NNEG_PAD

_mesh = plsc.VectorSubcoreMesh(core_axis_name="c", subcore_axis_name="s")
_sc_params = pltpu.CompilerParams(use_tc_tiling_on_sc=False)
_sc_params_nl = pltpu.CompilerParams(use_tc_tiling_on_sc=False,
                                     needs_layout_passes=False)


def _sc_degree_body(col_hbm, out_hbm, deg_sp, cidx_v, ones_v):
    cid = lax.axis_index("c")
    sid = lax.axis_index("s")

    def zfill(i, _):
        ones_v[pl.ds(i * 16, 16)] = jnp.zeros((16,), jnp.float32)
        return 0

    lax.fori_loop(0, RPT // 16, zfill, 0)
    pltpu.sync_copy(ones_v.at[pl.ds(0, RPT)], deg_sp.at[pl.ds(sid * RPT, RPT)])

    def fill(i, _):
        ones_v[pl.ds(i * 16, 16)] = jnp.ones((16,), jnp.float32)
        return 0

    lax.fori_loop(0, CHUNK_D // 16, fill, 0)
    plsc.subcore_barrier()

    epw = E // NW
    base = (cid * NS + sid) * epw

    def body(k, _):
        pltpu.sync_copy(col_hbm.at[pl.ds(base + k * CHUNK_D, CHUNK_D)], cidx_v)
        pltpu.sync_copy(ones_v, deg_sp.at[cidx_v], add=True)
        return 0

    lax.fori_loop(0, epw // CHUNK_D, body, 0)
    plsc.subcore_barrier()
    pltpu.sync_copy(deg_sp.at[pl.ds(sid * RPT, RPT)],
                    out_hbm.at[cid, pl.ds(sid * RPT, RPT)])


def _sc_degree(col):
    k = functools.partial(
        pl.kernel,
        out_type=jax.ShapeDtypeStruct((NC, NPAD), jnp.float32),
        mesh=_mesh,
        compiler_params=_sc_params,
        scratch_types=[
            pltpu.VMEM_SHARED((NPAD,), jnp.float32),
            pltpu.VMEM((CHUNK_D,), jnp.int32),
            pltpu.VMEM((CHUNK_D,), jnp.float32),
        ],
    )(_sc_degree_body)
    return k(col)


def _sc_edge_body(y_hbm, row2d_hbm, col2d_hbm, out_hbm,
                  y_sp, acc_sp, ridx_all, cidx_all,
                  rows_a, rows_b, sem_a, sem_b):
    cid = lax.axis_index("c")
    sid = lax.axis_index("s")
    fbase = cid * FH
    r0 = sid * RPT

    def zfill(i, _):
        rows_a[i // 2, pl.ds((i % 2) * 16, 16)] = jnp.zeros((16,), jnp.float32)
        return 0

    lax.fori_loop(0, CK * 2, zfill, 0)
    pltpu.sync_copy(rows_a, acc_sp.at[pl.ds(r0, CK)])
    pltpu.sync_copy(rows_a.at[pl.ds(0, RPT - CK)],
                    acc_sp.at[pl.ds(r0 + CK, RPT - CK)])
    pltpu.sync_copy(y_hbm.at[pl.ds(r0, RPT), pl.ds(fbase, FH)],
                    y_sp.at[pl.ds(r0, RPT)])
    pltpu.sync_copy(row2d_hbm.at[pl.ds(sid * CPT_P, CPT_P)], ridx_all)
    pltpu.sync_copy(col2d_hbm.at[pl.ds(sid * CPT_P, CPT_P)], cidx_all)
    plsc.subcore_barrier()

    pltpu.async_copy(y_sp.at[ridx_all.at[0]], rows_a, sem_a)

    def body(j, _):
        k0 = 2 * j
        pltpu.make_async_copy(y_sp.at[ridx_all.at[k0]], rows_a, sem_a).wait()
        pltpu.async_copy(y_sp.at[ridx_all.at[k0 + 1]], rows_b, sem_b)
        pltpu.sync_copy(rows_a, acc_sp.at[cidx_all.at[k0]], add=True)
        pltpu.make_async_copy(y_sp.at[ridx_all.at[k0 + 1]], rows_b,
                              sem_b).wait()

        @pl.when(k0 + 2 < CPT_P)
        def _():
            pltpu.async_copy(y_sp.at[ridx_all.at[k0 + 2]], rows_a, sem_a)

        pltpu.sync_copy(rows_b, acc_sp.at[cidx_all.at[k0 + 1]], add=True)
        return 0

    lax.fori_loop(0, CPT_P // 2, body, 0)
    plsc.subcore_barrier()
    pltpu.sync_copy(acc_sp.at[pl.ds(r0, RPT)],
                    out_hbm.at[pl.ds(r0, RPT), pl.ds(fbase, FH)])


def _sc_edge_pass(y_pad, row2d, col2d):
    k = functools.partial(
        pl.kernel,
        out_type=jax.ShapeDtypeStruct((NPAD, H), jnp.float32),
        mesh=_mesh,
        compiler_params=_sc_params,
        scratch_types=[
            pltpu.VMEM_SHARED((NPAD, FH), jnp.float32),
            pltpu.VMEM_SHARED((NPAD, FH), jnp.float32),
            pltpu.VMEM((CPT_P, CK), jnp.int32),
            pltpu.VMEM((CPT_P, CK), jnp.int32),
            pltpu.VMEM((CK, FH), jnp.float32),
            pltpu.VMEM((CK, FH), jnp.float32),
            pltpu.SemaphoreType.DMA,
            pltpu.SemaphoreType.DMA,
        ],
    )(_sc_edge_body)
    return k(y_pad, row2d, col2d)


def _sc_edge_final_body(y_hbm, row2d_hbm, col2d_hbm, dinv_hbm, b2_hbm,
                        rep_out,
                        y_sp, acc_sp, ridx_all, cidx_all,
                        rows_a, rows_b, dinv_v, b2h, sem_a, sem_b):
    cid = lax.axis_index("c")
    sid = lax.axis_index("s")
    fbase = cid * FH
    r0 = sid * RPT

    def zfill(i, _):
        rows_a[i // 2, pl.ds((i % 2) * 16, 16)] = jnp.zeros((16,), jnp.float32)
        return 0

    lax.fori_loop(0, CK * 2, zfill, 0)
    pltpu.sync_copy(rows_a, acc_sp.at[pl.ds(r0, CK)])
    pltpu.sync_copy(rows_a.at[pl.ds(0, RPT - CK)],
                    acc_sp.at[pl.ds(r0 + CK, RPT - CK)])
    pltpu.sync_copy(y_hbm.at[pl.ds(r0, RPT), pl.ds(fbase, FH)],
                    y_sp.at[pl.ds(r0, RPT)])
    pltpu.sync_copy(row2d_hbm.at[pl.ds(sid * CPT_P, CPT_P)], ridx_all)
    pltpu.sync_copy(col2d_hbm.at[pl.ds(sid * CPT_P, CPT_P)], cidx_all)
    pltpu.sync_copy(dinv_hbm.at[pl.ds(r0, RPT)], dinv_v.at[pl.ds(0, RPT)])
    pltpu.sync_copy(b2_hbm.at[pl.ds(fbase, FH)], b2h)
    plsc.subcore_barrier()

    pltpu.async_copy(y_sp.at[ridx_all.at[0]], rows_a, sem_a)

    def body(j, _):
        k0 = 2 * j
        pltpu.make_async_copy(y_sp.at[ridx_all.at[k0]], rows_a, sem_a).wait()
        pltpu.async_copy(y_sp.at[ridx_all.at[k0 + 1]], rows_b, sem_b)
        pltpu.sync_copy(rows_a, acc_sp.at[cidx_all.at[k0]], add=True)
        pltpu.make_async_copy(y_sp.at[ridx_all.at[k0 + 1]], rows_b,
                              sem_b).wait()

        @pl.when(k0 + 2 < CPT_P)
        def _():
            pltpu.async_copy(y_sp.at[ridx_all.at[k0 + 2]], rows_a, sem_a)

        pltpu.sync_copy(rows_b, acc_sp.at[cidx_all.at[k0 + 1]], add=True)
        return 0

    lax.fori_loop(0, CPT_P // 2, body, 0)
    plsc.subcore_barrier()

    b2v0 = b2h[pl.ds(0, 16)]
    b2v1 = b2h[pl.ds(16, 16)]

    def span(off, length):
        pltpu.sync_copy(acc_sp.at[pl.ds(r0 + off, length)],
                        rows_a.at[pl.ds(0, length)])
        pltpu.sync_copy(y_sp.at[pl.ds(r0 + off, length)],
                        rows_b.at[pl.ds(0, length)])

        def rw(r, _):
            dv = dinv_v[pl.ds(off + r, 16)][0]
            v0 = rows_a[r, pl.ds(0, 16)] + rows_b[r, pl.ds(0, 16)]
            v1 = rows_a[r, pl.ds(16, 16)] + rows_b[r, pl.ds(16, 16)]
            rows_a[r, pl.ds(0, 16)] = v0 * dv + b2v0
            rows_a[r, pl.ds(16, 16)] = v1 * dv + b2v1
            return 0

        lax.fori_loop(0, length, rw, 0)
        pltpu.sync_copy(rows_a.at[pl.ds(0, length)],
                        rep_out.at[pl.ds(r0 + off, length), pl.ds(fbase, FH)])

    span(0, CK)

    @pl.when(sid < NS - 1)
    def _():
        span(CK, RPT - CK)


def _sc_edge_final(y_pad, row2d, col2d, dinv1d, b2):
    k = functools.partial(
        pl.kernel,
        out_type=jax.ShapeDtypeStruct((N, H), jnp.float32),
        mesh=_mesh,
        compiler_params=_sc_params,
        scratch_types=[
            pltpu.VMEM_SHARED((NPAD, FH), jnp.float32),
            pltpu.VMEM_SHARED((NPAD, FH), jnp.float32),
            pltpu.VMEM((CPT_P, CK), jnp.int32),
            pltpu.VMEM((CPT_P, CK), jnp.int32),
            pltpu.VMEM((CK, FH), jnp.float32),
            pltpu.VMEM((CK, FH), jnp.float32),
            pltpu.VMEM((RPT + 16,), jnp.float32),
            pltpu.VMEM((FH,), jnp.float32),
            pltpu.SemaphoreType.DMA,
            pltpu.SemaphoreType.DMA,
        ],
    )(_sc_edge_final_body)
    return k(y_pad, row2d, col2d, dinv1d, b2)


def _sc_score_body(rep_hbm, pr_hbm, pc_hbm, nr_hbm, nc_hbm, out_hbm,
                   rep_sp, pr_idx, pc_idx, nr_idx, nc_idx,
                   ar0, br0, ar1, br1, d0, d1,
                   sem_g0, sem_g1, sem_w0, sem_w1):
    cid = lax.axis_index("c")
    sid = lax.axis_index("s")
    fbase = cid * FH
    r0 = sid * RPT

    rs0 = sid * (N // NS)
    pltpu.sync_copy(rep_hbm.at[pl.ds(rs0, N // NS), pl.ds(fbase, FH)],
                    rep_sp.at[pl.ds(rs0, N // NS)])
    pltpu.sync_copy(pr_hbm.at[pl.ds(sid * CPT_P, CPT_P)], pr_idx)
    pltpu.sync_copy(pc_hbm.at[pl.ds(sid * CPT_P, CPT_P)], pc_idx)
    pltpu.sync_copy(nr_hbm.at[pl.ds(sid * CPT_N, CPT_N)], nr_idx)
    pltpu.sync_copy(nc_hbm.at[pl.ds(sid * CPT_N, CPT_N)], nc_idx)
    plsc.subcore_barrier()

    lane = lax.iota(jnp.int32, 16)

    def compute(arows, brows, dbuf):
        def grp(g, _):
            rows0 = g * 16 + lane
            accs = [jnp.zeros((16,), jnp.float32) for _ in range(4)]
            for f in range(FH):
                colsf = jnp.bitwise_and(f + lane, FH - 1)
                a = plsc.load_gather(arows, [rows0, colsf])
                b = plsc.load_gather(brows, [rows0, colsf])
                accs[f % 4] = accs[f % 4] + a * b
            dbuf[pl.ds(g * 16, 16)] = (accs[0] + accs[1]) + (accs[2] + accs[3])
            return 0

        lax.fori_loop(0, CK // 16, grp, 0)

    def run(cpt, ridx, cidx, obase):
        def gather(t, ar, br, sem):
            pltpu.async_copy(rep_sp.at[ridx.at[t]], ar, sem)
            pltpu.async_copy(rep_sp.at[cidx.at[t]], br, sem)

        def drain(t, ar, br, sem):
            pltpu.make_async_copy(rep_sp.at[ridx.at[t]], ar, sem).wait()
            pltpu.make_async_copy(rep_sp.at[cidx.at[t]], br, sem).wait()

        gather(0, ar0, br0, sem_g0)

        def body(j, _):
            k0 = 2 * j
            drain(k0, ar0, br0, sem_g0)
            gather(k0 + 1, ar1, br1, sem_g1)

            @pl.when(j > 0)
            def _():
                pltpu.make_async_copy(
                    d0, out_hbm.at[cid, pl.ds(0, CK)], sem_w0).wait()

            compute(ar0, br0, d0)
            off0 = obase + (sid * cpt + k0) * CK
            pltpu.async_copy(d0, out_hbm.at[cid, pl.ds(off0, CK)], sem_w0)

            drain(k0 + 1, ar1, br1, sem_g1)

            @pl.when(k0 + 2 < cpt)
            def _():
                gather(k0 + 2, ar0, br0, sem_g0)

            @pl.when(j > 0)
            def _():
                pltpu.make_async_copy(
                    d1, out_hbm.at[cid, pl.ds(0, CK)], sem_w1).wait()

            compute(ar1, br1, d1)
            off1 = off0 + CK
            pltpu.async_copy(d1, out_hbm.at[cid, pl.ds(off1, CK)], sem_w1)
            return 0

        lax.fori_loop(0, cpt // 2, body, 0)
        pltpu.make_async_copy(d0, out_hbm.at[cid, pl.ds(0, CK)], sem_w0).wait()
        pltpu.make_async_copy(d1, out_hbm.at[cid, pl.ds(0, CK)], sem_w1).wait()

    run(CPT_P, pr_idx, pc_idx, 0)
    run(CPT_N, nr_idx, nc_idx, E)


def _sc_score(rep_pad, pr2d, pc2d, nr2d, nc2d):
    k = functools.partial(
        pl.kernel,
        out_type=jax.ShapeDtypeStruct((NC, ETOT), jnp.float32),
        mesh=_mesh,
        compiler_params=_sc_params_nl,
        scratch_types=[
            pltpu.VMEM_SHARED((N, FH), jnp.float32),
            pltpu.VMEM((CPT_P, CK), jnp.int32),
            pltpu.VMEM((CPT_P, CK), jnp.int32),
            pltpu.VMEM((CPT_N, CK), jnp.int32),
            pltpu.VMEM((CPT_N, CK), jnp.int32),
            pltpu.VMEM((CK, FH), jnp.float32),
            pltpu.VMEM((CK, FH), jnp.float32),
            pltpu.VMEM((CK, FH), jnp.float32),
            pltpu.VMEM((CK, FH), jnp.float32),
            pltpu.VMEM((CK,), jnp.float32),
            pltpu.VMEM((CK,), jnp.float32),
            pltpu.SemaphoreType.DMA,
            pltpu.SemaphoreType.DMA,
            pltpu.SemaphoreType.DMA,
            pltpu.SemaphoreType.DMA,
        ],
    )(_sc_score_body)
    return k(rep_pad, pr2d, pc2d, nr2d, nc2d)


def _prep_body(x_ref, w_ref, degt_ref, dinv_ref, y_ref):
    d = degt_ref[...]
    deg = d[:, 0:1] + d[:, 1:2] + 1.0
    dinv = lax.rsqrt(deg)
    dinv_ref[...] = dinv
    xw = jnp.dot(x_ref[...], w_ref[...], preferred_element_type=jnp.float32)
    y_ref[0:N, :] = dinv[0:N] * xw
    y_ref[N:NPAD, :] = jnp.zeros((NPAD - N, H), jnp.float32)


def _tc_prep(features, W1, degt):
    return pl.pallas_call(
        _prep_body,
        out_shape=[jax.ShapeDtypeStruct((NPAD, 1), jnp.float32),
                   jax.ShapeDtypeStruct((NPAD, H), jnp.float32)],
    )(features, W1, degt)


def _mid_body(acc_ref, y_ref, dinv_ref, b1_ref, w2_ref, o_ref):
    dinv = dinv_ref[...]
    s = acc_ref[...] + y_ref[...]
    h = jnp.maximum(dinv * s + b1_ref[...], 0.0)
    xw2 = jnp.dot(h, w2_ref[...], preferred_element_type=jnp.float32)
    y2 = dinv * xw2
    o_ref[0:N, :] = y2[0:N]
    o_ref[N:NPAD, :] = jnp.zeros((NPAD - N, H), jnp.float32)


def _tc_mid(acc1, y1p, dinvp, b1, W2):
    return pl.pallas_call(
        _mid_body,
        out_shape=jax.ShapeDtypeStruct((NPAD, H), jnp.float32),
    )(acc1, y1p, dinvp, b1, W2)


def _combine_body(dots_ref, pr_ref, pc_ref, nr_ref, nc_ref, o_ref):
    dp = dots_ref[0] + dots_ref[1]
    pos_d = dp[0:E // 128]
    neg_d = dp[E // 128:ETOT // 128]
    mp = (pr_ref[...] < pc_ref[...]).astype(jnp.float32)
    mn = (nr_ref[...] < nc_ref[...]).astype(jnp.float32)
    t = pos_d - 1.0
    s_pos = jnp.sum(mp * t * t)
    s_neg = jnp.sum(mn * neg_d * neg_d)
    denom = jnp.sum(mp) + jnp.sum(mn)
    rec = (s_neg + s_pos) * jnp.float32(N) / denom
    o_ref[...] = jnp.broadcast_to(rec, (1, 1))


def _tc_combine(dots3d, pr, pc, nr, nc):
    return pl.pallas_call(
        _combine_body,
        out_shape=jax.ShapeDtypeStruct((1, 1), jnp.float32),
    )(dots3d, pr, pc, nr, nc)


def kernel(features, edge_index, neg_edge_index, W1, b1, W2, b2):
    assert features.shape == (N, F_IN)
    assert edge_index.shape == (2, E)
    assert neg_edge_index.shape == (2, NNEG)

    row = edge_index[0]
    col = edge_index[1]
    nr = neg_edge_index[0]
    nc = neg_edge_index[1]
    zpad = jnp.zeros((NNEG_PAD - NNEG,), jnp.int32)
    nr_p = jnp.concatenate([nr, zpad])
    nc_p = jnp.concatenate([nc, zpad])
    row2d = row.reshape(NCH_P, CK)
    col2d = col.reshape(NCH_P, CK)
    nr2d = nr_p.reshape(NCH_N, CK)
    nc2d = nc_p.reshape(NCH_N, CK)

    degp = _sc_degree(col)
    degt = jnp.transpose(degp)
    dinvp, y1p = _tc_prep(features, W1, degt)
    acc1 = _sc_edge_pass(y1p, row2d, col2d)
    y2p = _tc_mid(acc1, y1p, dinvp, b1, W2)
    rep = _sc_edge_final(y2p, row2d, col2d, dinvp.reshape(NPAD), b2)
    dots = _sc_score(rep, row2d, col2d, nr2d, nc2d)
    dots3d = dots.reshape(NC, ETOT // 128, 128)
    rec_loss = _tc_combine(dots3d,
                           row.reshape(E // 128, 128),
                           col.reshape(E // 128, 128),
                           nr_p.reshape(NNEG_PAD // 128, 128),
                           nc_p.reshape(NNEG_PAD // 128, 128))[0, 0]
    return (rep, rec_loss)
